# baseline (device time: 213914 ns/iter reference)
import jax
import jax.numpy as jnp
from jax import lax
from jax.experimental import pallas as pl
from jax.experimental.pallas import tpu as pltpu

N_DEV = 16
F8 = jnp.float8_e4m3fn


def kernel(x, w_mat, scale_x, scale_w):
    m_per, k = x.shape
    _, n_per = w_mat.shape

    def body(x_ref, w_ref, sx_ref, sw_ref, out_ref,
             xg_ref, w8_ref, send_sems, recv_sems):
        my = lax.axis_index("i")
        left = (my - 1) % N_DEV
        right = (my + 1) % N_DEV

        barrier_sem = pltpu.get_barrier_semaphore()
        pl.semaphore_signal(barrier_sem, inc=1, device_id=(left,),
                            device_id_type=pl.DeviceIdType.MESH)
        pl.semaphore_signal(barrier_sem, inc=1, device_id=(right,),
                            device_id_type=pl.DeviceIdType.MESH)
        pl.semaphore_wait(barrier_sem, 2)

        w8_ref[...] = w_ref[...].astype(F8)
        xg_ref[0] = x_ref[...].astype(F8)
        scale = sx_ref[0] * sw_ref[0]

        def compute(slot, origin):
            acc = lax.dot_general(
                xg_ref[slot], w8_ref[...],
                (((1,), (0,)), ((), ())),
                preferred_element_type=jnp.float32)
            out_ref[pl.ds(origin * m_per, m_per), :] = (
                jnp.maximum(acc * scale, 0.0))

        compute(0, my)

        for h in range(N_DEV - 1):
            rdma = pltpu.make_async_remote_copy(
                src_ref=xg_ref.at[h],
                dst_ref=xg_ref.at[h + 1],
                send_sem=send_sems.at[h],
                recv_sem=recv_sems.at[h],
                device_id=(right,),
                device_id_type=pl.DeviceIdType.MESH,
            )
            rdma.start()
            rdma.wait()
            compute(h + 1, (my - h - 1) % N_DEV)

    return pl.pallas_call(
        body,
        out_shape=jax.ShapeDtypeStruct((N_DEV * m_per, n_per), jnp.float32),
        in_specs=[
            pl.BlockSpec(memory_space=pltpu.VMEM),
            pl.BlockSpec(memory_space=pltpu.VMEM),
            pl.BlockSpec(memory_space=pltpu.SMEM),
            pl.BlockSpec(memory_space=pltpu.SMEM),
        ],
        out_specs=pl.BlockSpec(memory_space=pltpu.VMEM),
        scratch_shapes=[
            pltpu.VMEM((N_DEV, m_per, k), F8),
            pltpu.VMEM((k, n_per), F8),
            pltpu.SemaphoreType.DMA((N_DEV - 1,)),
            pltpu.SemaphoreType.DMA((N_DEV - 1,)),
        ],
        compiler_params=pltpu.CompilerParams(
            collective_id=0,
            vmem_limit_bytes=100 * 1024 * 1024,
        ),
    )(x, w_mat, scale_x, scale_w)


# device time: 128535 ns/iter; 1.6642x vs baseline; 1.6642x over previous
import jax
import jax.numpy as jnp
from jax import lax
from jax.experimental import pallas as pl
from jax.experimental.pallas import tpu as pltpu

N_DEV = 16
R_HOPS = 8
L_HOPS = 7
F8 = jnp.float8_e4m3fn


def kernel(x, w_mat, scale_x, scale_w):
    m_per, k = x.shape
    _, n_per = w_mat.shape

    def body(x_ref, w_ref, sx_ref, sw_ref, out_ref,
             rbuf, lbuf, w8_ref, r_send, r_recv, l_send, l_recv):
        my = lax.axis_index("i")
        left = (my - 1) % N_DEV
        right = (my + 1) % N_DEV

        barrier_sem = pltpu.get_barrier_semaphore()
        pl.semaphore_signal(barrier_sem, inc=1, device_id=(left,),
                            device_id_type=pl.DeviceIdType.MESH)
        pl.semaphore_signal(barrier_sem, inc=1, device_id=(right,),
                            device_id_type=pl.DeviceIdType.MESH)
        pl.semaphore_wait(barrier_sem, 2)

        w8_ref[...] = w_ref[...].astype(F8)
        x8 = x_ref[...].astype(F8)
        rbuf[0] = x8
        lbuf[0] = x8
        scale = sx_ref[0] * sw_ref[0]

        def compute(buf, slot, origin):
            acc = lax.dot_general(
                buf[slot], w8_ref[...],
                (((1,), (0,)), ((), ())),
                preferred_element_type=jnp.float32)
            out_ref[pl.ds(origin * m_per, m_per), :] = (
                jnp.maximum(acc * scale, 0.0))

        def mk(buf, sends, recvs, h, dev):
            return pltpu.make_async_remote_copy(
                src_ref=buf.at[h], dst_ref=buf.at[h + 1],
                send_sem=sends.at[h], recv_sem=recvs.at[h],
                device_id=(dev,), device_id_type=pl.DeviceIdType.MESH)

        for h in range(R_HOPS):
            r = mk(rbuf, r_send, r_recv, h, right)
            r.start()
            l = mk(lbuf, l_send, l_recv, h, left) if h < L_HOPS else None
            if l is not None:
                l.start()

            if h == 0:
                compute(rbuf, 0, my)
            else:
                compute(rbuf, h, (my - h) % N_DEV)
                compute(lbuf, h, (my + h) % N_DEV)

            r.wait_recv()
            if l is not None:
                l.wait_recv()
            r.wait_send()
            if l is not None:
                l.wait_send()

        compute(rbuf, R_HOPS, (my - R_HOPS) % N_DEV)
        compute(lbuf, L_HOPS, (my + L_HOPS) % N_DEV)

    return pl.pallas_call(
        body,
        out_shape=jax.ShapeDtypeStruct((N_DEV * m_per, n_per), jnp.float32),
        in_specs=[
            pl.BlockSpec(memory_space=pltpu.VMEM),
            pl.BlockSpec(memory_space=pltpu.VMEM),
            pl.BlockSpec(memory_space=pltpu.SMEM),
            pl.BlockSpec(memory_space=pltpu.SMEM),
        ],
        out_specs=pl.BlockSpec(memory_space=pltpu.VMEM),
        scratch_shapes=[
            pltpu.VMEM((R_HOPS + 1, m_per, k), F8),
            pltpu.VMEM((L_HOPS + 1, m_per, k), F8),
            pltpu.VMEM((k, n_per), F8),
            pltpu.SemaphoreType.DMA((R_HOPS,)),
            pltpu.SemaphoreType.DMA((R_HOPS,)),
            pltpu.SemaphoreType.DMA((L_HOPS,)),
            pltpu.SemaphoreType.DMA((L_HOPS,)),
        ],
        compiler_params=pltpu.CompilerParams(
            collective_id=0,
            vmem_limit_bytes=100 * 1024 * 1024,
        ),
    )(x, w_mat, scale_x, scale_w)


# device time: 105646 ns/iter; 2.0248x vs baseline; 1.2167x over previous
import jax
import jax.numpy as jnp
from jax import lax
from jax.experimental import pallas as pl
from jax.experimental.pallas import tpu as pltpu

N_DEV = 16
R_HOPS = 8
L_HOPS = 7
S = 2
F8 = jnp.float8_e4m3fn


def kernel(x, w_mat, scale_x, scale_w):
    m_per, k = x.shape
    _, n_per = w_mat.shape
    m_sub = m_per // S

    def body(x_ref, w_ref, sx_ref, sw_ref, out_ref,
             rbuf, lbuf, w8_ref, r_send, r_recv, l_send, l_recv):
        my = lax.axis_index("i")
        left = (my - 1) % N_DEV
        right = (my + 1) % N_DEV

        barrier_sem = pltpu.get_barrier_semaphore()
        pl.semaphore_signal(barrier_sem, inc=1, device_id=(left,),
                            device_id_type=pl.DeviceIdType.MESH)
        pl.semaphore_signal(barrier_sem, inc=1, device_id=(right,),
                            device_id_type=pl.DeviceIdType.MESH)
        pl.semaphore_wait(barrier_sem, 2)

        w8_ref[...] = w_ref[...].astype(F8)
        x8 = x_ref[...].astype(F8)
        rbuf[0] = x8
        lbuf[0] = x8
        scale = sx_ref[0] * sw_ref[0]

        def compute_sub(buf, h, s, origin):
            acc = lax.dot_general(
                buf[h, pl.ds(s * m_sub, m_sub)], w8_ref[...],
                (((1,), (0,)), ((), ())),
                preferred_element_type=jnp.float32)
            out_ref[pl.ds(origin * m_per + s * m_sub, m_sub), :] = (
                jnp.maximum(acc * scale, 0.0))

        def start(buf, send_sems, recv_sems, h, s, dev):
            d = pltpu.make_async_remote_copy(
                src_ref=buf.at[h, pl.ds(s * m_sub, m_sub)],
                dst_ref=buf.at[h + 1, pl.ds(s * m_sub, m_sub)],
                send_sem=send_sems.at[s],
                recv_sem=recv_sems.at[h, s],
                device_id=(dev,), device_id_type=pl.DeviceIdType.MESH)
            d.start()
            return d

        rin = [start(rbuf, r_send, r_recv, 0, s, right) for s in range(S)]
        lin = [start(lbuf, l_send, l_recv, 0, s, left) for s in range(S)]
        for s in range(S):
            compute_sub(rbuf, 0, s, my)

        for h in range(1, R_HOPS + 1):
            for s in range(S):
                rin[s].wait_recv()
                if h < R_HOPS:
                    rin[s].wait_send()
                    rin[s] = start(rbuf, r_send, r_recv, h, s, right)
                compute_sub(rbuf, h, s, (my - h) % N_DEV)
            if h <= L_HOPS:
                for s in range(S):
                    lin[s].wait_recv()
                    if h < L_HOPS:
                        lin[s].wait_send()
                        lin[s] = start(lbuf, l_send, l_recv, h, s, left)
                    compute_sub(lbuf, h, s, (my + h) % N_DEV)

        for s in range(S):
            rin[s].wait_send()
            lin[s].wait_send()

    return pl.pallas_call(
        body,
        out_shape=jax.ShapeDtypeStruct((N_DEV * m_per, n_per), jnp.float32),
        in_specs=[
            pl.BlockSpec(memory_space=pltpu.VMEM),
            pl.BlockSpec(memory_space=pltpu.VMEM),
            pl.BlockSpec(memory_space=pltpu.SMEM),
            pl.BlockSpec(memory_space=pltpu.SMEM),
        ],
        out_specs=pl.BlockSpec(memory_space=pltpu.VMEM),
        scratch_shapes=[
            pltpu.VMEM((R_HOPS + 1, m_per, k), F8),
            pltpu.VMEM((L_HOPS + 1, m_per, k), F8),
            pltpu.VMEM((k, n_per), F8),
            pltpu.SemaphoreType.DMA((S,)),
            pltpu.SemaphoreType.DMA((R_HOPS, S)),
            pltpu.SemaphoreType.DMA((S,)),
            pltpu.SemaphoreType.DMA((L_HOPS, S)),
        ],
        compiler_params=pltpu.CompilerParams(
            collective_id=0,
            vmem_limit_bytes=100 * 1024 * 1024,
        ),
    )(x, w_mat, scale_x, scale_w)


# device time: 9107 ns/iter; 23.4890x vs baseline; 11.6005x over previous
import jax
import jax.numpy as jnp
from jax import lax
from jax.experimental import pallas as pl
from jax.experimental.pallas import tpu as pltpu

N_DEV = 16
HOPS = 8
S = 2
F8 = jnp.float8_e4m3fn

CHAIN_HOPS = ((8, 7), (7, 8))


def kernel(x, w_mat, scale_x, scale_w):
    m_per, k = x.shape
    _, n_per = w_mat.shape
    m_sub = m_per // S

    def body(x_ref, w_ref, sx_ref, sw_ref, out_ref,
             rbuf, lbuf, w8_ref, r_send, r_recv, l_send, l_recv):
        my = lax.axis_index("i")
        left = (my - 1) % N_DEV
        right = (my + 1) % N_DEV

        barrier_sem = pltpu.get_barrier_semaphore()
        pl.semaphore_signal(barrier_sem, inc=1, device_id=(left,),
                            device_id_type=pl.DeviceIdType.MESH)
        pl.semaphore_signal(barrier_sem, inc=1, device_id=(right,),
                            device_id_type=pl.DeviceIdType.MESH)
        pl.semaphore_wait(barrier_sem, 2)

        bufs = (rbuf, lbuf)
        send_sems = (r_send, l_send)
        recv_sems = (r_recv, l_recv)
        targets = (right, left)

        def start(d, h, s):
            src = (rbuf if h == 0 else bufs[d]).at[h, pl.ds(s * m_sub, m_sub)]
            r = pltpu.make_async_remote_copy(
                src_ref=src,
                dst_ref=bufs[d].at[h + 1, pl.ds(s * m_sub, m_sub)],
                send_sem=send_sems[d].at[s],
                recv_sem=recv_sems[d].at[h, s],
                device_id=(targets[d],), device_id_type=pl.DeviceIdType.MESH)
            r.start()
            return r

        def compute_sub(d, h, s, origin):
            acc = lax.dot_general(
                bufs[d][h, pl.ds(s * m_sub, m_sub)], w8_ref[...],
                (((1,), (0,)), ((), ())),
                preferred_element_type=jnp.float32)
            out_ref[pl.ds(origin * m_per + s * m_sub, m_sub), :] = (
                jnp.maximum(acc * scale, 0.0))

        inflight = [[None, None], [None, None]]
        for s in range(S):
            rbuf[0, pl.ds(s * m_sub, m_sub)] = (
                x_ref[pl.ds(s * m_sub, m_sub), :].astype(F8))
            for d in range(2):
                inflight[d][s] = start(d, 0, s)

        w8_ref[...] = w_ref[...].astype(F8)
        scale = sx_ref[0] * sw_ref[0]

        for s in range(S):
            compute_sub(0, 0, s, my)

        for h in range(1, HOPS + 1):
            for s in range(S):
                for d in range(2):
                    hops = CHAIN_HOPS[d][s]
                    if h > hops:
                        continue
                    inflight[d][s].wait_recv()
                    if h < hops:
                        inflight[d][s].wait_send()
                        inflight[d][s] = start(d, h, s)
                    origin = (my - h) % N_DEV if d == 0 else (my + h) % N_DEV
                    compute_sub(d, h, s, origin)

        for d in range(2):
            for s in range(S):
                inflight[d][s].wait_send()

    return pl.pallas_call(
        body,
        out_shape=jax.ShapeDtypeStruct((N_DEV * m_per, n_per), jnp.float32),
        in_specs=[
            pl.BlockSpec(memory_space=pltpu.VMEM),
            pl.BlockSpec(memory_space=pltpu.VMEM),
            pl.BlockSpec(memory_space=pltpu.SMEM),
            pl.BlockSpec(memory_space=pltpu.SMEM),
        ],
        out_specs=pl.BlockSpec(memory_space=pltpu.VMEM),
        scratch_shapes=[
            pltpu.VMEM((HOPS + 1, m_per, k), F8),
            pltpu.VMEM((HOPS + 1, m_per, k), F8),
            pltpu.VMEM((k, n_per), F8),
            pltpu.SemaphoreType.DMA((S,)),
            pltpu.SemaphoreType.DMA((HOPS, S)),
            pltpu.SemaphoreType.DMA((S,)),
            pltpu.SemaphoreType.DMA((HOPS, S)),
        ],
        compiler_params=pltpu.CompilerParams(
            collective_id=0,
            vmem_limit_bytes=100 * 1024 * 1024,
        ),
    )(x, w_mat, scale_x, scale_w)
